# Initial kernel scaffold; baseline (speedup 1.0000x reference)
import jax
import jax.numpy as jnp
from jax.experimental import pallas as pl


def kernel(x, knn_db, Wq, Wkv, Wo, scale_param, output_gate):
    raise NotImplementedError("write your pallas kernel here")


# speedup vs baseline: 7.5016x; 7.5016x over previous
import functools

import jax
import jax.numpy as jnp
from jax.experimental import pallas as pl

DH = 64
TOPK = 32


def _proj_q_kernel(x_ref, wq_ref, qn_ref):
    q = jax.lax.dot(x_ref[...], wq_ref[...],
                    preferred_element_type=jnp.float32)
    for j in range(2):
        qj = q[:, j * DH:(j + 1) * DH]
        norm = jnp.sqrt(jnp.sum(qj * qj, axis=-1, keepdims=True))
        qn_ref[j] = qj / jnp.maximum(norm, 1e-12)


def _proj_kv_kernel(x_ref, wkv_ref, kn_ref, v_ref):
    kv = jax.lax.dot(x_ref[...], wkv_ref[...],
                     preferred_element_type=jnp.float32)
    k = kv[:, :DH]
    norm = jnp.sqrt(jnp.sum(k * k, axis=-1, keepdims=True))
    kn_ref[...] = k / jnp.maximum(norm, 1e-12)
    v_ref[...] = kv[:, DH:]


def _attn_kernel(scale_ref, qn_ref, kn_ref, v_ref, db_ref, o_ref, *, bq, n):
    qb = qn_ref[0]
    sc = jnp.exp(scale_ref[0, 0, 0])
    db = db_ref[...]
    db_k = db[:, :DH]
    db_v = db[:, DH:]

    lm = jax.lax.dot_general(qb, db_k, (((1,), (1,)), ((), ())),
                             preferred_element_type=jnp.float32)
    lm = lm * sc - sc

    def body(_, w):
        m = jnp.max(w, axis=-1, keepdims=True)
        return jnp.where(w == m, -jnp.inf, w)

    work = jax.lax.fori_loop(0, TOPK, body, lm)
    topk_mask = work == -jnp.inf
    u = jnp.where(topk_mask, jnp.exp(lm), 0.0)
    z_mem = jnp.sum(u, axis=-1, keepdims=True)
    num_mem = jax.lax.dot(u, db_v, preferred_element_type=jnp.float32)

    sl = jax.lax.dot_general(qb, kn_ref[...], (((1,), (1,)), ((), ())),
                             preferred_element_type=jnp.float32)
    sl = sl * sc - sc
    i = pl.program_id(1)
    q_pos = i * bq + jax.lax.broadcasted_iota(jnp.int32, (bq, n), 0)
    k_pos = jax.lax.broadcasted_iota(jnp.int32, (bq, n), 1)
    p = jnp.where(k_pos <= q_pos, jnp.exp(sl), 0.0)
    z_loc = jnp.sum(p, axis=-1, keepdims=True)
    num_loc = jax.lax.dot(p, v_ref[...], preferred_element_type=jnp.float32)

    o_ref[0] = (num_mem + num_loc) / (z_mem + z_loc)


def _out_kernel(x_ref, o_ref, wo_ref, gate_ref, out_ref, *, h):
    hh = pl.program_id(0)
    contrib = jax.lax.dot(o_ref[0], wo_ref[...],
                          preferred_element_type=jnp.float32)

    @pl.when(hh == 0)
    def _():
        out_ref[...] = contrib

    @pl.when(hh > 0)
    def _():
        out_ref[...] += contrib

    @pl.when(hh == h - 1)
    def _():
        out_ref[...] = x_ref[...] + out_ref[...] * jnp.tanh(gate_ref[0, 0])


def kernel(x, knn_db, Wq, Wkv, Wo, scale_param, output_gate):
    b, n, dim = x.shape
    h = Wq.shape[1] // DH
    m = knn_db.shape[1]
    x2 = x.reshape(n, dim)
    db = knn_db.reshape(m, 2 * DH)

    qn = pl.pallas_call(
        _proj_q_kernel,
        grid=(h // 2,),
        in_specs=[
            pl.BlockSpec((n, dim), lambda i: (0, 0)),
            pl.BlockSpec((dim, 2 * DH), lambda i: (0, i)),
        ],
        out_specs=pl.BlockSpec((2, n, DH), lambda i: (i, 0, 0)),
        out_shape=jax.ShapeDtypeStruct((h, n, DH), jnp.float32),
    )(x2, Wq)

    kn, v = pl.pallas_call(
        _proj_kv_kernel,
        out_shape=[jax.ShapeDtypeStruct((n, DH), jnp.float32),
                   jax.ShapeDtypeStruct((n, DH), jnp.float32)],
    )(x2, Wkv)

    bq = 256
    nb = n // bq
    o_heads = pl.pallas_call(
        functools.partial(_attn_kernel, bq=bq, n=n),
        grid=(h, nb),
        in_specs=[
            pl.BlockSpec((1, 1, 1), lambda hh, i: (hh, 0, 0)),
            pl.BlockSpec((1, bq, DH), lambda hh, i: (hh, i, 0)),
            pl.BlockSpec((n, DH), lambda hh, i: (0, 0)),
            pl.BlockSpec((n, DH), lambda hh, i: (0, 0)),
            pl.BlockSpec((m, 2 * DH), lambda hh, i: (0, 0)),
        ],
        out_specs=pl.BlockSpec((1, bq, DH), lambda hh, i: (hh, i, 0)),
        out_shape=jax.ShapeDtypeStruct((h, n, DH), jnp.float32),
    )(scale_param, qn, kn, v, db)

    out = pl.pallas_call(
        functools.partial(_out_kernel, h=h),
        grid=(h,),
        in_specs=[
            pl.BlockSpec((n, dim), lambda hh: (0, 0)),
            pl.BlockSpec((1, n, DH), lambda hh: (hh, 0, 0)),
            pl.BlockSpec((DH, dim), lambda hh: (hh, 0)),
            pl.BlockSpec((1, 1), lambda hh: (0, 0)),
        ],
        out_specs=pl.BlockSpec((n, dim), lambda hh: (0, 0)),
        out_shape=jax.ShapeDtypeStruct((n, dim), jnp.float32),
    )(x2, o_heads, Wo, output_gate.reshape(1, 1))

    return out.reshape(b, n, dim)


# speedup vs baseline: 19.5891x; 2.6113x over previous
import functools

import jax
import jax.numpy as jnp
from jax.experimental import pallas as pl

DH = 64
TOPK = 32


def _proj_q_kernel(x_ref, wq_ref, qn_ref):
    q = jax.lax.dot(x_ref[...], wq_ref[...],
                    preferred_element_type=jnp.float32)
    for j in range(2):
        qj = q[:, j * DH:(j + 1) * DH]
        norm = jnp.sqrt(jnp.sum(qj * qj, axis=-1, keepdims=True))
        qn_ref[j] = qj / jnp.maximum(norm, 1e-12)


def _proj_kv_kernel(x_ref, wkv_ref, kn_ref, v_ref):
    kv = jax.lax.dot(x_ref[...], wkv_ref[...],
                     preferred_element_type=jnp.float32)
    k = kv[:, :DH]
    norm = jnp.sqrt(jnp.sum(k * k, axis=-1, keepdims=True))
    kn_ref[...] = k / jnp.maximum(norm, 1e-12)
    v_ref[...] = kv[:, DH:]


def _attn_kernel(scale_ref, qn_ref, kn_ref, v_ref, db_ref, o_ref, *, bq, n):
    sc = jnp.exp(scale_ref[0, 0, 0])
    qb = qn_ref[0] * sc
    db = db_ref[...]
    db_k = db[:, :DH]
    db_v = db[:, DH:]

    lm = jax.lax.dot_general(qb, db_k, (((1,), (1,)), ((), ())),
                             preferred_element_type=jnp.float32)

    mm = lm.shape[1]
    work = lm.reshape(bq, 32, mm // 32)
    depth = 4
    gms = []
    for j in range(depth):
        gm = jnp.max(work, axis=1, keepdims=True)
        gms.append(gm)
        if j < depth - 1:
            work = jnp.where(work == gm, -jnp.inf, work)
    cand = jnp.concatenate(gms, axis=1).reshape(bq, depth * (mm // 32))

    def extract_t(arr):
        def body(_, carry):
            w, _ = carry
            m = jnp.max(w, axis=-1, keepdims=True)
            return jnp.where(w == m, -jnp.inf, w), m
        _, t = jax.lax.fori_loop(
            0, TOPK, body, (arr, jnp.zeros((bq, 1), jnp.float32)))
        return t

    t = extract_t(cand)
    ge = lm >= t
    cnt = jnp.sum(jnp.where(ge, 1.0, 0.0), axis=-1, keepdims=True)

    def fast_u():
        return jnp.where(ge, jnp.exp(lm), 0.0)

    def slow_u():
        return jnp.where(lm >= extract_t(lm), jnp.exp(lm), 0.0)

    u = jax.lax.cond(jnp.any(cnt != TOPK), slow_u, fast_u)
    z_mem = jnp.sum(u, axis=-1, keepdims=True)
    num_mem = jax.lax.dot(u, db_v, preferred_element_type=jnp.float32)

    sl = jax.lax.dot_general(qb, kn_ref[...], (((1,), (1,)), ((), ())),
                             preferred_element_type=jnp.float32)
    i = pl.program_id(1)
    q_pos = i * bq + jax.lax.broadcasted_iota(jnp.int32, (bq, n), 0)
    k_pos = jax.lax.broadcasted_iota(jnp.int32, (bq, n), 1)
    p = jnp.where(k_pos <= q_pos, jnp.exp(sl), 0.0)
    z_loc = jnp.sum(p, axis=-1, keepdims=True)
    num_loc = jax.lax.dot(p, v_ref[...], preferred_element_type=jnp.float32)

    o_ref[0] = (num_mem + num_loc) / (z_mem + z_loc)


def _out_kernel(x_ref, o_ref, wo_ref, gate_ref, out_ref, *, h):
    hh = pl.program_id(0)
    contrib = jax.lax.dot(o_ref[0], wo_ref[...],
                          preferred_element_type=jnp.float32)

    @pl.when(hh == 0)
    def _():
        out_ref[...] = contrib

    @pl.when(hh > 0)
    def _():
        out_ref[...] += contrib

    @pl.when(hh == h - 1)
    def _():
        out_ref[...] = x_ref[...] + out_ref[...] * jnp.tanh(gate_ref[0, 0])


def kernel(x, knn_db, Wq, Wkv, Wo, scale_param, output_gate):
    b, n, dim = x.shape
    h = Wq.shape[1] // DH
    m = knn_db.shape[1]
    x2 = x.reshape(n, dim)
    db = knn_db.reshape(m, 2 * DH)

    qn = pl.pallas_call(
        _proj_q_kernel,
        grid=(h // 2,),
        in_specs=[
            pl.BlockSpec((n, dim), lambda i: (0, 0)),
            pl.BlockSpec((dim, 2 * DH), lambda i: (0, i)),
        ],
        out_specs=pl.BlockSpec((2, n, DH), lambda i: (i, 0, 0)),
        out_shape=jax.ShapeDtypeStruct((h, n, DH), jnp.float32),
    )(x2, Wq)

    kn, v = pl.pallas_call(
        _proj_kv_kernel,
        out_shape=[jax.ShapeDtypeStruct((n, DH), jnp.float32),
                   jax.ShapeDtypeStruct((n, DH), jnp.float32)],
    )(x2, Wkv)

    bq = 256
    nb = n // bq
    o_heads = pl.pallas_call(
        functools.partial(_attn_kernel, bq=bq, n=n),
        grid=(h, nb),
        in_specs=[
            pl.BlockSpec((1, 1, 1), lambda hh, i: (hh, 0, 0)),
            pl.BlockSpec((1, bq, DH), lambda hh, i: (hh, i, 0)),
            pl.BlockSpec((n, DH), lambda hh, i: (0, 0)),
            pl.BlockSpec((n, DH), lambda hh, i: (0, 0)),
            pl.BlockSpec((m, 2 * DH), lambda hh, i: (0, 0)),
        ],
        out_specs=pl.BlockSpec((1, bq, DH), lambda hh, i: (hh, i, 0)),
        out_shape=jax.ShapeDtypeStruct((h, n, DH), jnp.float32),
    )(scale_param, qn, kn, v, db)

    out = pl.pallas_call(
        functools.partial(_out_kernel, h=h),
        grid=(h,),
        in_specs=[
            pl.BlockSpec((n, dim), lambda hh: (0, 0)),
            pl.BlockSpec((1, n, DH), lambda hh: (hh, 0, 0)),
            pl.BlockSpec((DH, dim), lambda hh: (hh, 0)),
            pl.BlockSpec((1, 1), lambda hh: (0, 0)),
        ],
        out_specs=pl.BlockSpec((n, dim), lambda hh: (0, 0)),
        out_shape=jax.ShapeDtypeStruct((n, dim), jnp.float32),
    )(x2, o_heads, Wo, output_gate.reshape(1, 1))

    return out.reshape(b, n, dim)


# speedup vs baseline: 26.6808x; 1.3620x over previous
import functools

import jax
import jax.numpy as jnp
from jax.experimental import pallas as pl

DH = 64
TOPK = 32


def _proj_q_kernel(x_ref, wq_ref, qn_ref):
    q = jax.lax.dot(x_ref[...], wq_ref[...],
                    preferred_element_type=jnp.float32)
    for j in range(2):
        qj = q[:, j * DH:(j + 1) * DH]
        norm = jnp.sqrt(jnp.sum(qj * qj, axis=-1, keepdims=True))
        qn_ref[j] = qj / jnp.maximum(norm, 1e-12)


def _proj_kv_kernel(x_ref, wkv_ref, kn_ref, v_ref):
    kv = jax.lax.dot(x_ref[...], wkv_ref[...],
                     preferred_element_type=jnp.float32)
    k = kv[:, :DH]
    norm = jnp.sqrt(jnp.sum(k * k, axis=-1, keepdims=True))
    kn_ref[...] = k / jnp.maximum(norm, 1e-12)
    v_ref[...] = kv[:, DH:]


def _attn_kernel(scale_ref, qn_ref, kn_ref, v_ref, db_ref, o_ref, *, bq, n):
    sc = jnp.exp(scale_ref[0, 0, 0])
    qb = qn_ref[0] * sc
    db = db_ref[...]
    db_k = db[:, :DH]
    db_v = db[:, DH:]

    lm = jax.lax.dot_general(qb, db_k, (((1,), (1,)), ((), ())),
                             preferred_element_type=jnp.float32)

    mm = lm.shape[1]
    ns = mm // 256
    sl = [lm[:, j * 256:(j + 1) * 256] for j in range(ns)]
    s2 = [(jnp.maximum(sl[2 * i], sl[2 * i + 1]),
           jnp.minimum(sl[2 * i], sl[2 * i + 1])) for i in range(ns // 2)]
    s4 = []
    for i in range(ns // 4):
        (x0, x1), (y0, y1) = s2[2 * i], s2[2 * i + 1]
        z0 = jnp.maximum(x0, y0)
        tt = jnp.minimum(x0, y0)
        z3 = jnp.minimum(x1, y1)
        ss = jnp.maximum(x1, y1)
        s4.append((z0, jnp.maximum(tt, ss), jnp.minimum(tt, ss), z3))

    def merge4(x, y, sort):
        m = [jnp.maximum(x[i], y[3 - i]) for i in range(4)]
        if not sort:
            return m
        a0, a2 = jnp.maximum(m[0], m[2]), jnp.minimum(m[0], m[2])
        a1, a3 = jnp.maximum(m[1], m[3]), jnp.minimum(m[1], m[3])
        return (jnp.maximum(a0, a1), jnp.minimum(a0, a1),
                jnp.maximum(a2, a3), jnp.minimum(a2, a3))

    while len(s4) > 1:
        s4 = [merge4(s4[2 * i], s4[2 * i + 1], sort=len(s4) > 2)
              for i in range(len(s4) // 2)]
    cand = jnp.concatenate(s4[0], axis=-1)

    def extract_t(arr):
        def body(_, carry):
            w, _ = carry
            m = jnp.max(w, axis=-1, keepdims=True)
            return jnp.where(w == m, -jnp.inf, w), m
        _, t = jax.lax.fori_loop(
            0, TOPK, body, (arr, jnp.zeros((bq, 1), jnp.float32)))
        return t

    t = extract_t(cand)
    cnt = jnp.sum(jnp.where(lm >= t, 1.0, 0.0), axis=-1, keepdims=True)
    t = jax.lax.cond(jnp.any(cnt != TOPK), lambda: extract_t(lm), lambda: t)
    u = jnp.where(lm >= t, jnp.exp(lm), 0.0)
    z_mem = jnp.sum(u, axis=-1, keepdims=True)
    num_mem = jax.lax.dot(u, db_v, preferred_element_type=jnp.float32)

    sl = jax.lax.dot_general(qb, kn_ref[...], (((1,), (1,)), ((), ())),
                             preferred_element_type=jnp.float32)
    i = pl.program_id(1)
    q_pos = i * bq + jax.lax.broadcasted_iota(jnp.int32, (bq, n), 0)
    k_pos = jax.lax.broadcasted_iota(jnp.int32, (bq, n), 1)
    p = jnp.where(k_pos <= q_pos, jnp.exp(sl), 0.0)
    z_loc = jnp.sum(p, axis=-1, keepdims=True)
    num_loc = jax.lax.dot(p, v_ref[...], preferred_element_type=jnp.float32)

    o_ref[0] = (num_mem + num_loc) / (z_mem + z_loc)


def _out_kernel(x_ref, o_ref, wo_ref, gate_ref, out_ref, *, h):
    hh = pl.program_id(0)
    contrib = jax.lax.dot(o_ref[0], wo_ref[...],
                          preferred_element_type=jnp.float32)

    @pl.when(hh == 0)
    def _():
        out_ref[...] = contrib

    @pl.when(hh > 0)
    def _():
        out_ref[...] += contrib

    @pl.when(hh == h - 1)
    def _():
        out_ref[...] = x_ref[...] + out_ref[...] * jnp.tanh(gate_ref[0, 0])


def kernel(x, knn_db, Wq, Wkv, Wo, scale_param, output_gate):
    b, n, dim = x.shape
    h = Wq.shape[1] // DH
    m = knn_db.shape[1]
    x2 = x.reshape(n, dim)
    db = knn_db.reshape(m, 2 * DH)

    qn = pl.pallas_call(
        _proj_q_kernel,
        grid=(h // 2,),
        in_specs=[
            pl.BlockSpec((n, dim), lambda i: (0, 0)),
            pl.BlockSpec((dim, 2 * DH), lambda i: (0, i)),
        ],
        out_specs=pl.BlockSpec((2, n, DH), lambda i: (i, 0, 0)),
        out_shape=jax.ShapeDtypeStruct((h, n, DH), jnp.float32),
    )(x2, Wq)

    kn, v = pl.pallas_call(
        _proj_kv_kernel,
        out_shape=[jax.ShapeDtypeStruct((n, DH), jnp.float32),
                   jax.ShapeDtypeStruct((n, DH), jnp.float32)],
    )(x2, Wkv)

    bq = 256
    nb = n // bq
    o_heads = pl.pallas_call(
        functools.partial(_attn_kernel, bq=bq, n=n),
        grid=(h, nb),
        in_specs=[
            pl.BlockSpec((1, 1, 1), lambda hh, i: (hh, 0, 0)),
            pl.BlockSpec((1, bq, DH), lambda hh, i: (hh, i, 0)),
            pl.BlockSpec((n, DH), lambda hh, i: (0, 0)),
            pl.BlockSpec((n, DH), lambda hh, i: (0, 0)),
            pl.BlockSpec((m, 2 * DH), lambda hh, i: (0, 0)),
        ],
        out_specs=pl.BlockSpec((1, bq, DH), lambda hh, i: (hh, i, 0)),
        out_shape=jax.ShapeDtypeStruct((h, n, DH), jnp.float32),
    )(scale_param, qn, kn, v, db)

    out = pl.pallas_call(
        functools.partial(_out_kernel, h=h),
        grid=(h,),
        in_specs=[
            pl.BlockSpec((n, dim), lambda hh: (0, 0)),
            pl.BlockSpec((1, n, DH), lambda hh: (hh, 0, 0)),
            pl.BlockSpec((DH, dim), lambda hh: (hh, 0)),
            pl.BlockSpec((1, 1), lambda hh: (0, 0)),
        ],
        out_specs=pl.BlockSpec((n, dim), lambda hh: (0, 0)),
        out_shape=jax.ShapeDtypeStruct((n, dim), jnp.float32),
    )(x2, o_heads, Wo, output_gate.reshape(1, 1))

    return out.reshape(b, n, dim)


# speedup vs baseline: 43.8403x; 1.6431x over previous
import functools

import jax
import jax.numpy as jnp
from jax.experimental import pallas as pl

DH = 64
TOPK = 32


def _proj_q_kernel(x_ref, wq_ref, qn_ref):
    q = jax.lax.dot(x_ref[...], wq_ref[...],
                    preferred_element_type=jnp.float32)
    for j in range(2):
        qj = q[:, j * DH:(j + 1) * DH]
        norm = jnp.sqrt(jnp.sum(qj * qj, axis=-1, keepdims=True))
        qn_ref[j] = qj / jnp.maximum(norm, 1e-12)


def _proj_kv_kernel(x_ref, wkv_ref, kn_ref, v_ref):
    kv = jax.lax.dot(x_ref[...], wkv_ref[...],
                     preferred_element_type=jnp.float32)
    k = kv[:, :DH]
    norm = jnp.sqrt(jnp.sum(k * k, axis=-1, keepdims=True))
    kn_ref[...] = k / jnp.maximum(norm, 1e-12)
    v_ref[...] = kv[:, DH:]


def _attn_kernel(scale_ref, qn_ref, kn_ref, va_ref, dbk_ref, dbva_ref,
                 o_ref, *, bq, n):
    sc = jnp.exp(scale_ref[0, 0, 0])
    qb = qn_ref[0] * sc

    lm = jax.lax.dot_general(qb, dbk_ref[...], (((1,), (1,)), ((), ())),
                             preferred_element_type=jnp.float32)

    mm = lm.shape[1]
    ns = mm // 256
    sl = [lm[:, j * 256:(j + 1) * 256] for j in range(ns)]
    s2 = [(jnp.maximum(sl[2 * i], sl[2 * i + 1]),
           jnp.minimum(sl[2 * i], sl[2 * i + 1])) for i in range(ns // 2)]
    s4 = []
    for i in range(ns // 4):
        (x0, x1), (y0, y1) = s2[2 * i], s2[2 * i + 1]
        z0 = jnp.maximum(x0, y0)
        tt = jnp.minimum(x0, y0)
        z3 = jnp.minimum(x1, y1)
        ss = jnp.maximum(x1, y1)
        s4.append((z0, jnp.maximum(tt, ss), jnp.minimum(tt, ss), z3))

    def merge4(x, y, sort):
        m = [jnp.maximum(x[i], y[3 - i]) for i in range(4)]
        if not sort:
            return m
        a0, a2 = jnp.maximum(m[0], m[2]), jnp.minimum(m[0], m[2])
        a1, a3 = jnp.maximum(m[1], m[3]), jnp.minimum(m[1], m[3])
        return (jnp.maximum(a0, a1), jnp.minimum(a0, a1),
                jnp.maximum(a2, a3), jnp.minimum(a2, a3))

    while len(s4) > 1:
        s4 = [merge4(s4[2 * i], s4[2 * i + 1], sort=True)
              for i in range(len(s4) // 2)]
    c0, c1, c2, c3 = s4[0]

    def extract_t(arr):
        def body(_, carry):
            w, _ = carry
            m = jnp.max(w, axis=-1, keepdims=True)
            return jnp.where(w == m, -jnp.inf, w), m
        _, t = jax.lax.fori_loop(
            0, TOPK, body, (arr, jnp.zeros((bq, 1), jnp.float32)))
        return t

    w0, w1, w2, w3 = c0, c1, c2, c3
    t = None
    for r in range(TOPK):
        t = jnp.max(w0, axis=-1, keepdims=True)
        if r < TOPK - 1:
            hit = w0 == t
            w0 = jnp.where(hit, w1, w0)
            w1 = jnp.where(hit, w2, w1)
            w2 = jnp.where(hit, w3, w2)
            w3 = jnp.where(hit, -jnp.inf, w3)
    cnt = jnp.sum(jnp.where(lm >= t, 1.0, 0.0), axis=-1, keepdims=True)
    t = jax.lax.cond(jnp.any(cnt != TOPK), lambda: extract_t(lm), lambda: t)
    u = jnp.where(lm >= t, jnp.exp(lm), 0.0).astype(jnp.bfloat16)
    nm = jax.lax.dot(u, dbva_ref[...], preferred_element_type=jnp.float32)
    num_mem, z_mem = nm[:, :DH], nm[:, DH:DH + 1]

    sl = jax.lax.dot_general(qb, kn_ref[...], (((1,), (1,)), ((), ())),
                             preferred_element_type=jnp.float32)
    i = pl.program_id(1)
    q_pos = i * bq + jax.lax.broadcasted_iota(jnp.int32, (bq, n), 0)
    k_pos = jax.lax.broadcasted_iota(jnp.int32, (bq, n), 1)
    p = jnp.where(k_pos <= q_pos, jnp.exp(sl), 0.0).astype(jnp.bfloat16)
    nl = jax.lax.dot(p, va_ref[...], preferred_element_type=jnp.float32)
    num_loc, z_loc = nl[:, :DH], nl[:, DH:DH + 1]

    o_ref[0] = ((num_mem + num_loc) / (z_mem + z_loc)).astype(o_ref.dtype)


def _out_kernel(x_ref, o_ref, wo_ref, gate_ref, out_ref, *, h):
    hh = pl.program_id(0)
    contrib = jax.lax.dot(o_ref[0], wo_ref[...],
                          preferred_element_type=jnp.float32)

    @pl.when(hh == 0)
    def _():
        out_ref[...] = contrib

    @pl.when(hh > 0)
    def _():
        out_ref[...] += contrib

    @pl.when(hh == h - 1)
    def _():
        out_ref[...] = x_ref[...] + out_ref[...] * jnp.tanh(gate_ref[0, 0])


def kernel(x, knn_db, Wq, Wkv, Wo, scale_param, output_gate):
    b, n, dim = x.shape
    h = Wq.shape[1] // DH
    m = knn_db.shape[1]
    x2 = x.reshape(n, dim)
    dbk = knn_db[:, :, 0, :].reshape(m, DH)
    dbva = jnp.concatenate(
        [knn_db[:, :, 1, :].reshape(m, DH), jnp.ones((m, 1), jnp.float32)],
        axis=1).astype(jnp.bfloat16)

    xb = x2.astype(jnp.bfloat16)
    qn = pl.pallas_call(
        _proj_q_kernel,
        grid=(h // 2,),
        in_specs=[
            pl.BlockSpec((n, dim), lambda i: (0, 0)),
            pl.BlockSpec((dim, 2 * DH), lambda i: (0, i)),
        ],
        out_specs=pl.BlockSpec((2, n, DH), lambda i: (i, 0, 0)),
        out_shape=jax.ShapeDtypeStruct((h, n, DH), jnp.float32),
    )(xb, Wq.astype(jnp.bfloat16))

    kn, v = pl.pallas_call(
        _proj_kv_kernel,
        out_shape=[jax.ShapeDtypeStruct((n, DH), jnp.float32),
                   jax.ShapeDtypeStruct((n, DH), jnp.float32)],
    )(xb, Wkv.astype(jnp.bfloat16))

    va = jnp.concatenate(
        [v, jnp.ones((n, 1), jnp.float32)], axis=1).astype(jnp.bfloat16)

    bq = min(256, n)
    nb = n // bq
    o_heads = pl.pallas_call(
        functools.partial(_attn_kernel, bq=bq, n=n),
        grid=(h, nb),
        in_specs=[
            pl.BlockSpec((1, 1, 1), lambda hh, i: (hh, 0, 0)),
            pl.BlockSpec((1, bq, DH), lambda hh, i: (hh, i, 0)),
            pl.BlockSpec((n, DH), lambda hh, i: (0, 0)),
            pl.BlockSpec((n, DH + 1), lambda hh, i: (0, 0)),
            pl.BlockSpec((m, DH), lambda hh, i: (0, 0)),
            pl.BlockSpec((m, DH + 1), lambda hh, i: (0, 0)),
        ],
        out_specs=pl.BlockSpec((1, bq, DH), lambda hh, i: (hh, i, 0)),
        out_shape=jax.ShapeDtypeStruct((h, n, DH), jnp.bfloat16),
    )(scale_param, qn, kn, va, dbk, dbva)

    out = pl.pallas_call(
        functools.partial(_out_kernel, h=h),
        grid=(h,),
        in_specs=[
            pl.BlockSpec((n, dim), lambda hh: (0, 0)),
            pl.BlockSpec((1, n, DH), lambda hh: (hh, 0, 0)),
            pl.BlockSpec((DH, dim), lambda hh: (hh, 0)),
            pl.BlockSpec((1, 1), lambda hh: (0, 0)),
        ],
        out_specs=pl.BlockSpec((n, dim), lambda hh: (0, 0)),
        out_shape=jax.ShapeDtypeStruct((n, dim), jnp.float32),
    )(x2, o_heads, Wo.astype(jnp.bfloat16), output_gate.reshape(1, 1))

    return out.reshape(b, n, dim)


# speedup vs baseline: 44.0099x; 1.0039x over previous
import functools

import jax
import jax.numpy as jnp
from jax.experimental import pallas as pl

DH = 64
TOPK = 32


def _proj_q_kernel(x_ref, wq_ref, qn_ref):
    q = jax.lax.dot(x_ref[...], wq_ref[...],
                    preferred_element_type=jnp.float32)
    for j in range(2):
        qj = q[:, j * DH:(j + 1) * DH]
        norm = jnp.sqrt(jnp.sum(qj * qj, axis=-1, keepdims=True))
        qn_ref[j] = qj / jnp.maximum(norm, 1e-12)


def _proj_kv_kernel(x_ref, wkv_ref, kn_ref, v_ref):
    kv = jax.lax.dot(x_ref[...], wkv_ref[...],
                     preferred_element_type=jnp.float32)
    k = kv[:, :DH]
    norm = jnp.sqrt(jnp.sum(k * k, axis=-1, keepdims=True))
    kn_ref[...] = k / jnp.maximum(norm, 1e-12)
    v_ref[...] = kv[:, DH:]


def _attn_kernel(scale_ref, qn_ref, kn_ref, va_ref, dbk_ref, dbva_ref,
                 o_ref, *, bq, n):
    sc = jnp.exp(scale_ref[0, 0, 0])
    qb = qn_ref[0] * sc

    lm = jax.lax.dot_general(qb, dbk_ref[...], (((1,), (1,)), ((), ())),
                             preferred_element_type=jnp.float32)

    mm = lm.shape[1]
    ns = mm // 256
    sl = [lm[:, j * 256:(j + 1) * 256] for j in range(ns)]
    s2 = [(jnp.maximum(sl[2 * i], sl[2 * i + 1]),
           jnp.minimum(sl[2 * i], sl[2 * i + 1])) for i in range(ns // 2)]
    s4 = []
    for i in range(ns // 4):
        (x0, x1), (y0, y1) = s2[2 * i], s2[2 * i + 1]
        z0 = jnp.maximum(x0, y0)
        tt = jnp.minimum(x0, y0)
        z3 = jnp.minimum(x1, y1)
        ss = jnp.maximum(x1, y1)
        s4.append((z0, jnp.maximum(tt, ss), jnp.minimum(tt, ss), z3))

    def merge4(x, y, sort):
        m = [jnp.maximum(x[i], y[3 - i]) for i in range(4)]
        if not sort:
            return m
        a0, a2 = jnp.maximum(m[0], m[2]), jnp.minimum(m[0], m[2])
        a1, a3 = jnp.maximum(m[1], m[3]), jnp.minimum(m[1], m[3])
        return (jnp.maximum(a0, a1), jnp.minimum(a0, a1),
                jnp.maximum(a2, a3), jnp.minimum(a2, a3))

    while len(s4) > 1:
        s4 = [merge4(s4[2 * i], s4[2 * i + 1], sort=True)
              for i in range(len(s4) // 2)]
    c0, c1, c2, c3 = s4[0]

    def extract_t(arr):
        def body(_, carry):
            w, _ = carry
            m = jnp.max(w, axis=-1, keepdims=True)
            return jnp.where(w == m, -jnp.inf, w), m
        _, t = jax.lax.fori_loop(
            0, TOPK, body, (arr, jnp.zeros((bq, 1), jnp.float32)))
        return t

    w0, w1, w2, w3 = c0, c1, c2, c3
    t = None
    for r in range(TOPK):
        t = jnp.max(w0, axis=-1, keepdims=True)
        if r < TOPK - 1:
            hit = w0 == t
            w0 = jnp.where(hit, w1, w0)
            w1 = jnp.where(hit, w2, w1)
            w2 = jnp.where(hit, w3, w2)
            w3 = jnp.where(hit, -jnp.inf, w3)
    cnt = jnp.sum(jnp.where(lm >= t, 1.0, 0.0), axis=-1, keepdims=True)
    t = jax.lax.cond(jnp.any(cnt != TOPK), lambda: extract_t(lm), lambda: t)
    u = jnp.where(lm >= t, jnp.exp(lm), 0.0).astype(jnp.bfloat16)
    nm = jax.lax.dot(u, dbva_ref[...], preferred_element_type=jnp.float32)
    num_mem, z_mem = nm[:, :DH], nm[:, DH:DH + 1]

    sl = jax.lax.dot_general(qb, kn_ref[...], (((1,), (1,)), ((), ())),
                             preferred_element_type=jnp.float32)
    i = pl.program_id(1)
    q_pos = i * bq + jax.lax.broadcasted_iota(jnp.int32, (bq, n), 0)
    k_pos = jax.lax.broadcasted_iota(jnp.int32, (bq, n), 1)
    p = jnp.where(k_pos <= q_pos, jnp.exp(sl), 0.0).astype(jnp.bfloat16)
    nl = jax.lax.dot(p, va_ref[...], preferred_element_type=jnp.float32)
    num_loc, z_loc = nl[:, :DH], nl[:, DH:DH + 1]

    o_ref[0] = ((num_mem + num_loc) / (z_mem + z_loc)).astype(o_ref.dtype)


def _out_kernel(x_ref, o_ref, wo_ref, gate_ref, out_ref, *, h):
    hh = pl.program_id(0)
    contrib = jax.lax.dot(o_ref[0], wo_ref[...],
                          preferred_element_type=jnp.float32)

    @pl.when(hh == 0)
    def _():
        out_ref[...] = contrib

    @pl.when(hh > 0)
    def _():
        out_ref[...] += contrib

    @pl.when(hh == h - 1)
    def _():
        out_ref[...] = x_ref[...] + out_ref[...] * jnp.tanh(gate_ref[0, 0])


def kernel(x, knn_db, Wq, Wkv, Wo, scale_param, output_gate):
    b, n, dim = x.shape
    h = Wq.shape[1] // DH
    m = knn_db.shape[1]
    x2 = x.reshape(n, dim)
    dbk = knn_db[:, :, 0, :].reshape(m, DH)
    dbva = jnp.concatenate(
        [knn_db[:, :, 1, :].reshape(m, DH), jnp.ones((m, 1), jnp.float32)],
        axis=1).astype(jnp.bfloat16)

    qn = pl.pallas_call(
        _proj_q_kernel,
        grid=(h // 2,),
        in_specs=[
            pl.BlockSpec((n, dim), lambda i: (0, 0)),
            pl.BlockSpec((dim, 2 * DH), lambda i: (0, i)),
        ],
        out_specs=pl.BlockSpec((2, n, DH), lambda i: (i, 0, 0)),
        out_shape=jax.ShapeDtypeStruct((h, n, DH), jnp.float32),
    )(x2, Wq)

    kn, v = pl.pallas_call(
        _proj_kv_kernel,
        out_shape=[jax.ShapeDtypeStruct((n, DH), jnp.float32),
                   jax.ShapeDtypeStruct((n, DH), jnp.float32)],
    )(x2, Wkv)

    va = jnp.concatenate(
        [v, jnp.ones((n, 1), jnp.float32)], axis=1).astype(jnp.bfloat16)

    bq = min(256, n)
    nb = n // bq
    o_heads = pl.pallas_call(
        functools.partial(_attn_kernel, bq=bq, n=n),
        grid=(h, nb),
        in_specs=[
            pl.BlockSpec((1, 1, 1), lambda hh, i: (hh, 0, 0)),
            pl.BlockSpec((1, bq, DH), lambda hh, i: (hh, i, 0)),
            pl.BlockSpec((n, DH), lambda hh, i: (0, 0)),
            pl.BlockSpec((n, DH + 1), lambda hh, i: (0, 0)),
            pl.BlockSpec((m, DH), lambda hh, i: (0, 0)),
            pl.BlockSpec((m, DH + 1), lambda hh, i: (0, 0)),
        ],
        out_specs=pl.BlockSpec((1, bq, DH), lambda hh, i: (hh, i, 0)),
        out_shape=jax.ShapeDtypeStruct((h, n, DH), jnp.float32),
    )(scale_param, qn, kn, va, dbk, dbva)

    out = pl.pallas_call(
        functools.partial(_out_kernel, h=h),
        grid=(h,),
        in_specs=[
            pl.BlockSpec((n, dim), lambda hh: (0, 0)),
            pl.BlockSpec((1, n, DH), lambda hh: (hh, 0, 0)),
            pl.BlockSpec((DH, dim), lambda hh: (hh, 0)),
            pl.BlockSpec((1, 1), lambda hh: (0, 0)),
        ],
        out_specs=pl.BlockSpec((n, dim), lambda hh: (0, 0)),
        out_shape=jax.ShapeDtypeStruct((n, dim), jnp.float32),
    )(x2, o_heads, Wo, output_gate.reshape(1, 1))

    return out.reshape(b, n, dim)
